# speedup vs baseline: 1.1374x; 1.1374x over previous
import jax
import jax.numpy as jnp
from jax import lax
from jax.experimental import pallas as pl
from jax.experimental.pallas import tpu as pltpu
from jax.experimental.pallas import tpu_sc as plsc

SEQ = 2048
DIM = 256
NC = 2
NS = 16
NW = NC * NS
TOTAL = 4 * SEQ
ROWS_PER_W = TOTAL // NW
CH = 128
NCHUNK = ROWS_PER_W // CH
LANES = 16
DCHUNKS = DIM // LANES


def _emb_body(x_hbm, tok_hbm, pos_hbm, out_hbm, idx_v, tok_v, pos_v, sem):
    wid = lax.axis_index("s") * NC + lax.axis_index("c")
    base = wid * ROWS_PER_W
    pos0 = lax.rem(base, SEQ)
    for j in range(NCHUNK):
        row0 = base + j * CH
        pltpu.sync_copy(x_hbm.at[pl.ds(row0, CH)], idx_v.at[j])
        gat = pltpu.async_copy(tok_hbm.at[idx_v.at[j]], tok_v, sem)
        pltpu.sync_copy(pos_hbm.at[pl.ds(pos0 + j * CH, CH)], pos_v)
        gat.wait()

        def body(r, carry):
            for c in range(DCHUNKS):
                sl = pl.ds(c * LANES, LANES)
                tok_v[r, sl] = tok_v[r, sl] + pos_v[r, sl]
            return carry

        lax.fori_loop(0, CH, body, 0)
        pltpu.sync_copy(tok_v, out_hbm.at[pl.ds(row0, CH)])


def kernel(x, token_table, pos_table):
    B, S = x.shape
    xf = x.reshape(B * S).astype(jnp.int32)
    call = pl.kernel(
        _emb_body,
        out_type=jax.ShapeDtypeStruct((B * S, DIM), jnp.float32),
        mesh=plsc.VectorSubcoreMesh(core_axis_name="c", subcore_axis_name="s"),
        scratch_types=[
            pltpu.VMEM((NCHUNK, CH), jnp.int32),
            pltpu.VMEM((CH, DIM), jnp.float32),
            pltpu.VMEM((CH, DIM), jnp.float32),
            pltpu.SemaphoreType.DMA,
        ],
    )
    out = call(xf, token_table, pos_table)
    return out.reshape(B, S, DIM)
